# baseline (speedup 1.0000x reference)
import functools

import jax
import jax.numpy as jnp
from jax import lax
from jax.experimental import pallas as pl
from jax.experimental.pallas import tpu as pltpu
from jax.experimental.pallas import tpu_sc as plsc

_NUM_STYLES = 64
_DIM = 192
_DA = 128
_DB = 64
_BATCH = 16384
_LANES = 16


def _table_body(emb_ref, w_ref, b_ref, p_ref):
    p_ref[...] = (
        lax.dot_general(
            emb_ref[...],
            w_ref[...],
            (((1,), (1,)), ((), ())),
            preferred_element_type=jnp.float32,
        )
        + b_ref[...]
    )


def _make_sc_kernel():
    info = plsc.get_sparse_core_info()
    nc, ns = info.num_cores, info.num_subcores
    nw = nc * ns
    bpw = _BATCH // nw
    nch = 4
    ch = bpw // nch
    ngb = 3
    na = _DA // _LANES
    nb = _DB // _LANES

    mesh = plsc.VectorSubcoreMesh(core_axis_name="c", subcore_axis_name="s")

    @functools.partial(
        pl.kernel,
        mesh=mesh,
        compiler_params=pltpu.CompilerParams(
            needs_layout_passes=False, use_tc_tiling_on_sc=False
        ),
        out_type=(
            jax.ShapeDtypeStruct((_BATCH * _DA,), jnp.float32),
            jax.ShapeDtypeStruct((_BATCH * _DB,), jnp.float32),
        ),
        scratch_types=[
            pltpu.VMEM((nch, ch), jnp.int32),
            pltpu.VMEM((bpw,), jnp.float32),
            pltpu.VMEM((ngb, ch, _DIM), jnp.float32),
            pltpu.VMEM((2, ch * _DA), jnp.float32),
            pltpu.VMEM((2, ch * _DB), jnp.float32),
            pltpu.SemaphoreType.DMA,
            pltpu.SemaphoreType.DMA,
            pltpu.SemaphoreType.DMA,
            pltpu.SemaphoreType.DMA,
            pltpu.SemaphoreType.DMA,
        ],
    )
    def sc_kernel(
        sid_hbm, exa_hbm, p_hbm, outa_hbm, outb_hbm,
        idx_v, exa_v, gbuf, abuf, bbuf, g0, g1, g2, o0, o1,
    ):
        wid = lax.axis_index("s") * nc + lax.axis_index("c")
        base = wid * bpw
        gsems = (g0, g1, g2)
        osems = (o0, o1)
        pltpu.sync_copy(sid_hbm.at[wid], idx_v)
        gathers = [
            pltpu.async_copy(p_hbm.at[idx_v.at[k]], gbuf.at[k], gsems[k])
            for k in range(ngb)
        ]
        pltpu.sync_copy(exa_hbm.at[wid], exa_v)

        stores = [None, None]
        for k in range(nch):
            s = k % 2
            g = k % ngb
            gathers[k].wait()
            if stores[s] is not None:
                for st in stores[s]:
                    st.wait()

            def body(r, _):
                e = plsc.load_gather(
                    exa_v, [jnp.full((_LANES,), k * ch + r, jnp.int32)]
                )
                for j in range(na):
                    src = gbuf[g, r, pl.ds(j * _LANES, _LANES)]
                    abuf[s, pl.ds(r * _DA + j * _LANES, _LANES)] = src * e
                for j in range(nb):
                    src = gbuf[g, r, pl.ds(_DA + j * _LANES, _LANES)]
                    bbuf[s, pl.ds(r * _DB + j * _LANES, _LANES)] = src * e
                return _

            lax.fori_loop(0, ch, body, 0, unroll=2)
            if k + ngb < nch:
                gathers.append(
                    pltpu.async_copy(
                        p_hbm.at[idx_v.at[k + ngb]], gbuf.at[g], gsems[g]
                    )
                )
            stores[s] = (
                pltpu.async_copy(
                    abuf.at[s],
                    outa_hbm.at[pl.ds((base + k * ch) * _DA, ch * _DA)],
                    osems[s],
                ),
                pltpu.async_copy(
                    bbuf.at[s],
                    outb_hbm.at[pl.ds((base + k * ch) * _DB, ch * _DB)],
                    osems[s],
                ),
            )
        for pair in stores:
            for st in pair:
                st.wait()

    return sc_kernel, nw, nch, ch


_SC_KERNEL, _NW, _NCH, _CH = _make_sc_kernel()


def kernel(style_id, exaggeration, emb, W, b):
    p = pl.pallas_call(
        _table_body,
        out_shape=jax.ShapeDtypeStruct((_NUM_STYLES, _DIM), jnp.float32),
    )(emb, W, b.reshape(1, _DIM))
    sid = style_id.reshape(_NW, _NCH, _CH)
    exa = exaggeration.reshape(_NW, _NCH * _CH)
    outa, outb = _SC_KERNEL(sid, exa, p)
    return jnp.concatenate(
        [outa.reshape(_BATCH, _DA), outb.reshape(_BATCH, _DB)], axis=1
    )
